# baseline (speedup 1.0000x reference)
import functools

import jax
import jax.numpy as jnp
from jax import lax
from jax.experimental import pallas as pl
from jax.experimental.pallas import tpu as pltpu
from jax.experimental.pallas import tpu_sc as plsc

B, L, D = 4, 4096, 1024
CHUNK = 8
UNROLL = 8


def _sc_add(x2, table):
    info = plsc.get_sparse_core_info()
    nc, ns = info.num_cores, info.num_subcores
    nw = nc * ns
    rows_w = L // nw
    nch = rows_w // CHUNK

    mesh = plsc.VectorSubcoreMesh(core_axis_name="c", subcore_axis_name="s")

    scratch = (
        [pltpu.VMEM((CHUNK, D), jnp.float32) for _ in range(2 * B)]
        + [pltpu.VMEM((CHUNK, D), jnp.float32) for _ in range(2)]
        + [pltpu.SemaphoreType.DMA for _ in range(2 * B)]
        + [pltpu.SemaphoreType.DMA for _ in range(2 * B)]
        + [pltpu.SemaphoreType.DMA for _ in range(2)]
    )

    @functools.partial(
        pl.kernel,
        mesh=mesh,
        out_type=jax.ShapeDtypeStruct((B * L, D), jnp.float32),
        scratch_types=scratch,
    )
    def k(x_hbm, t_hbm, o_hbm, *s):
        xbuf = [[s[2 * b + p] for p in range(2)] for b in range(B)]
        tbuf = [s[2 * B + p] for p in range(2)]
        base = 2 * B + 2
        xin = [[s[base + 2 * b + p] for p in range(2)] for b in range(B)]
        base += 2 * B
        xout = [[s[base + 2 * b + p] for p in range(2)] for b in range(B)]
        base += 2 * B
        tin = [s[base + p] for p in range(2)]

        w = lax.axis_index("s") * nc + lax.axis_index("c")
        row0 = w * rows_w

        def trow(c):
            return row0 + c * CHUNK

        def start_xload(c, b):
            pltpu.async_copy(
                x_hbm.at[pl.ds(b * L + trow(c), CHUNK), :],
                xbuf[b][c % 2], xin[b][c % 2])

        def start_tload(c):
            pltpu.async_copy(
                t_hbm.at[pl.ds(trow(c), CHUNK), :], tbuf[c % 2], tin[c % 2])

        def start_xstore(c, b):
            pltpu.async_copy(
                xbuf[b][c % 2],
                o_hbm.at[pl.ds(b * L + trow(c), CHUNK), :], xout[b][c % 2])

        def wait_xin(c, b):
            pltpu.make_async_copy(
                x_hbm.at[pl.ds(0, CHUNK), :], xbuf[b][c % 2],
                xin[b][c % 2]).wait()

        def wait_xout(c, b):
            pltpu.make_async_copy(
                xbuf[b][c % 2], o_hbm.at[pl.ds(0, CHUNK), :],
                xout[b][c % 2]).wait()

        def wait_tin(c):
            pltpu.make_async_copy(
                t_hbm.at[pl.ds(0, CHUNK), :], tbuf[c % 2], tin[c % 2]).wait()

        start_tload(0)
        start_tload(1)
        for b in range(B):
            start_xload(0, b)
            start_xload(1, b)

        for c in range(nch):
            p = c % 2
            if 1 <= c <= nch - 2:
                for b in range(B):
                    wait_xout(c - 1, b)
                    start_xload(c + 1, b)

            wait_tin(c)
            for b in range(B):
                wait_xin(c, b)

            tb = tbuf[p]
            xbs = [xbuf[b][p] for b in range(B)]

            nvec = CHUNK * D // 16
            cpr = D // 16

            @plsc.parallel_loop(0, nvec, 1, unroll=UNROLL)
            def _(j, tb=tb, xbs=xbs):
                i = jax.lax.div(j, cpr)
                sl = pl.ds(jax.lax.rem(j, cpr) * 16, 16)
                tv = tb[i, sl]
                for xb in xbs:
                    xb[i, sl] = xb[i, sl] + tv

            for b in range(B):
                start_xstore(c, b)
            if c + 2 < nch:
                start_tload(c + 2)

        for b in range(B):
            wait_xout(nch - 2, b)
            wait_xout(nch - 1, b)

    return k(x2, table)


def kernel(x, table):
    out = _sc_add(x.reshape(B * L, D), table)
    return out.reshape(B, L, D)


# speedup vs baseline: 1.0028x; 1.0028x over previous
import functools

import jax
import jax.numpy as jnp
from jax import lax
from jax.experimental import pallas as pl
from jax.experimental.pallas import tpu as pltpu
from jax.experimental.pallas import tpu_sc as plsc

B, L, D = 4, 4096, 1024
CHUNK = 8
UNROLL = 8


def _sc_add(x, table):
    info = plsc.get_sparse_core_info()
    nc, ns = info.num_cores, info.num_subcores
    nw = nc * ns
    rows_w = L // nw
    nch = rows_w // CHUNK

    mesh = plsc.VectorSubcoreMesh(core_axis_name="c", subcore_axis_name="s")

    scratch = (
        [pltpu.VMEM((CHUNK, D), jnp.float32) for _ in range(2 * B)]
        + [pltpu.VMEM((CHUNK, D), jnp.float32) for _ in range(2)]
        + [pltpu.SemaphoreType.DMA for _ in range(2 * B)]
        + [pltpu.SemaphoreType.DMA for _ in range(2 * B)]
        + [pltpu.SemaphoreType.DMA for _ in range(2)]
    )

    @functools.partial(
        pl.kernel,
        mesh=mesh,
        out_type=jax.ShapeDtypeStruct((B, L, D), jnp.float32),
        scratch_types=scratch,
    )
    def k(x_hbm, t_hbm, o_hbm, *s):
        xbuf = [[s[2 * b + p] for p in range(2)] for b in range(B)]
        tbuf = [s[2 * B + p] for p in range(2)]
        base = 2 * B + 2
        xin = [[s[base + 2 * b + p] for p in range(2)] for b in range(B)]
        base += 2 * B
        xout = [[s[base + 2 * b + p] for p in range(2)] for b in range(B)]
        base += 2 * B
        tin = [s[base + p] for p in range(2)]

        w = lax.axis_index("s") * nc + lax.axis_index("c")
        row0 = w * rows_w

        def trow(c):
            return row0 + c * CHUNK

        def start_xload(c, b):
            pltpu.async_copy(
                x_hbm.at[b, pl.ds(trow(c), CHUNK), :],
                xbuf[b][c % 2], xin[b][c % 2])

        def start_tload(c):
            pltpu.async_copy(
                t_hbm.at[pl.ds(trow(c), CHUNK), :], tbuf[c % 2], tin[c % 2])

        def start_xstore(c, b):
            pltpu.async_copy(
                xbuf[b][c % 2],
                o_hbm.at[b, pl.ds(trow(c), CHUNK), :], xout[b][c % 2])

        def wait_xin(c, b):
            pltpu.make_async_copy(
                x_hbm.at[0, pl.ds(0, CHUNK), :], xbuf[b][c % 2],
                xin[b][c % 2]).wait()

        def wait_xout(c, b):
            pltpu.make_async_copy(
                xbuf[b][c % 2], o_hbm.at[0, pl.ds(0, CHUNK), :],
                xout[b][c % 2]).wait()

        def wait_tin(c):
            pltpu.make_async_copy(
                t_hbm.at[pl.ds(0, CHUNK), :], tbuf[c % 2], tin[c % 2]).wait()

        start_tload(0)
        start_tload(1)
        for b in range(B):
            start_xload(0, b)
            start_xload(1, b)

        for c in range(nch):
            p = c % 2
            if 1 <= c <= nch - 2:
                for b in range(B):
                    wait_xout(c - 1, b)
                    start_xload(c + 1, b)

            wait_tin(c)
            for b in range(B):
                wait_xin(c, b)

            tb = tbuf[p]
            xbs = [xbuf[b][p] for b in range(B)]

            nvec = CHUNK * D // 16
            cpr = D // 16

            @plsc.parallel_loop(0, nvec, 1, unroll=UNROLL)
            def _(j, tb=tb, xbs=xbs):
                i = jax.lax.div(j, cpr)
                sl = pl.ds(jax.lax.rem(j, cpr) * 16, 16)
                tv = tb[i, sl]
                for xb in xbs:
                    xb[i, sl] = xb[i, sl] + tv

            for b in range(B):
                start_xstore(c, b)
            if c + 2 < nch:
                start_tload(c + 2)

        for b in range(B):
            wait_xout(nch - 2, b)
            wait_xout(nch - 1, b)

    return k(x, table)


def kernel(x, table):
    return _sc_add(x, table)


# speedup vs baseline: 1.0285x; 1.0255x over previous
import functools

import jax
import jax.numpy as jnp
from jax import lax
from jax.experimental import pallas as pl
from jax.experimental.pallas import tpu as pltpu
from jax.experimental.pallas import tpu_sc as plsc

B, L, D = 4, 4096, 1024
CHUNK = 8
UNROLL = 8


def _sc_add(x, table):
    info = plsc.get_sparse_core_info()
    nc, ns = info.num_cores, info.num_subcores
    nw = nc * ns
    rows_w = L // nw
    nch = rows_w // CHUNK

    mesh = plsc.VectorSubcoreMesh(core_axis_name="c", subcore_axis_name="s")

    scratch = (
        [pltpu.VMEM((B, CHUNK, D), jnp.float32) for _ in range(2)]
        + [pltpu.VMEM((CHUNK, D), jnp.float32) for _ in range(2)]
        + [pltpu.SemaphoreType.DMA for _ in range(2)]
        + [pltpu.SemaphoreType.DMA for _ in range(2)]
        + [pltpu.SemaphoreType.DMA for _ in range(2)]
    )

    @functools.partial(
        pl.kernel,
        mesh=mesh,
        out_type=jax.ShapeDtypeStruct((B, L, D), jnp.float32),
        scratch_types=scratch,
    )
    def k(x_hbm, t_hbm, o_hbm, xb0, xb1, tb0, tb1, si0, si1, so0, so1,
          st0, st1):
        xbuf = [xb0, xb1]
        tbuf = [tb0, tb1]
        xin = [si0, si1]
        xout = [so0, so1]
        tin = [st0, st1]

        w = lax.axis_index("s") * nc + lax.axis_index("c")
        row0 = w * rows_w

        def trow(c):
            return row0 + c * CHUNK

        def start_xload(c):
            pltpu.async_copy(
                x_hbm.at[:, pl.ds(trow(c), CHUNK), :], xbuf[c % 2],
                xin[c % 2])

        def start_tload(c):
            pltpu.async_copy(
                t_hbm.at[pl.ds(trow(c), CHUNK), :], tbuf[c % 2], tin[c % 2])

        def start_xstore(c):
            pltpu.async_copy(
                xbuf[c % 2], o_hbm.at[:, pl.ds(trow(c), CHUNK), :],
                xout[c % 2])

        def wait_xin(c):
            pltpu.make_async_copy(
                x_hbm.at[:, pl.ds(0, CHUNK), :], xbuf[c % 2],
                xin[c % 2]).wait()

        def wait_xout(c):
            pltpu.make_async_copy(
                xbuf[c % 2], o_hbm.at[:, pl.ds(0, CHUNK), :],
                xout[c % 2]).wait()

        def wait_tin(c):
            pltpu.make_async_copy(
                t_hbm.at[pl.ds(0, CHUNK), :], tbuf[c % 2], tin[c % 2]).wait()

        start_tload(0)
        start_tload(1)
        start_xload(0)
        start_xload(1)

        nvec = CHUNK * D // 16
        cpr = D // 16

        for c in range(nch):
            p = c % 2
            if 1 <= c <= nch - 2:
                wait_xout(c - 1)
                start_xload(c + 1)

            wait_tin(c)
            wait_xin(c)

            tb = tbuf[p]
            xb = xbuf[p]

            @plsc.parallel_loop(0, nvec, 1, unroll=UNROLL)
            def _(j, tb=tb, xb=xb):
                i = jax.lax.div(j, cpr)
                sl = pl.ds(jax.lax.rem(j, cpr) * 16, 16)
                tv = tb[i, sl]
                for b in range(B):
                    xb[b, i, sl] = xb[b, i, sl] + tv

            start_xstore(c)
            if c + 2 < nch:
                start_tload(c + 2)

        wait_xout(nch - 2)
        wait_xout(nch - 1)

    return k(x, table)


def kernel(x, table):
    return _sc_add(x, table)


# speedup vs baseline: 1.0383x; 1.0096x over previous
import functools

import jax
import jax.numpy as jnp
from jax import lax
from jax.experimental import pallas as pl
from jax.experimental.pallas import tpu as pltpu
from jax.experimental.pallas import tpu_sc as plsc

B, L, D = 4, 4096, 1024
CHUNK = 8
NBUF = 3
UNROLL = 8


def _sc_add(x, table):
    info = plsc.get_sparse_core_info()
    nc, ns = info.num_cores, info.num_subcores
    nw = nc * ns
    rows_w = L // nw
    nch = rows_w // CHUNK

    mesh = plsc.VectorSubcoreMesh(core_axis_name="c", subcore_axis_name="s")

    scratch = (
        [pltpu.VMEM((B, CHUNK, D), jnp.float32) for _ in range(NBUF)]
        + [pltpu.VMEM((CHUNK, D), jnp.float32) for _ in range(NBUF)]
        + [pltpu.SemaphoreType.DMA for _ in range(3 * NBUF)]
    )

    @functools.partial(
        pl.kernel,
        mesh=mesh,
        out_type=jax.ShapeDtypeStruct((B, L, D), jnp.float32),
        scratch_types=scratch,
    )
    def k(x_hbm, t_hbm, o_hbm, *s):
        xbuf = list(s[:NBUF])
        tbuf = list(s[NBUF:2 * NBUF])
        sems = list(s[2 * NBUF:])
        xin = sems[:NBUF]
        xout = sems[NBUF:2 * NBUF]
        tin = sems[2 * NBUF:]

        w = lax.axis_index("s") * nc + lax.axis_index("c")
        row0 = w * rows_w

        def trow(c):
            return row0 + c * CHUNK

        def start_xload(c):
            pltpu.async_copy(
                x_hbm.at[:, pl.ds(trow(c), CHUNK), :], xbuf[c % NBUF],
                xin[c % NBUF])

        def start_tload(c):
            pltpu.async_copy(
                t_hbm.at[pl.ds(trow(c), CHUNK), :], tbuf[c % NBUF],
                tin[c % NBUF])

        def start_xstore(c):
            pltpu.async_copy(
                xbuf[c % NBUF], o_hbm.at[:, pl.ds(trow(c), CHUNK), :],
                xout[c % NBUF])

        def wait_xin(c):
            pltpu.make_async_copy(
                x_hbm.at[:, pl.ds(0, CHUNK), :], xbuf[c % NBUF],
                xin[c % NBUF]).wait()

        def wait_xout(c):
            pltpu.make_async_copy(
                xbuf[c % NBUF], o_hbm.at[:, pl.ds(0, CHUNK), :],
                xout[c % NBUF]).wait()

        def wait_tin(c):
            pltpu.make_async_copy(
                t_hbm.at[pl.ds(0, CHUNK), :], tbuf[c % NBUF],
                tin[c % NBUF]).wait()

        for c0 in range(NBUF):
            start_tload(c0)
            start_xload(c0)

        nvec = CHUNK * D // 16
        cpr = D // 16

        for c in range(nch):
            if c >= NBUF - 1 and c + 1 < nch:
                wait_xout(c + 1 - NBUF)
                start_xload(c + 1)

            wait_tin(c)
            wait_xin(c)

            tb = tbuf[c % NBUF]
            xb = xbuf[c % NBUF]

            @plsc.parallel_loop(0, nvec, 1, unroll=UNROLL)
            def _(j, tb=tb, xb=xb):
                i = jax.lax.div(j, cpr)
                sl = pl.ds(jax.lax.rem(j, cpr) * 16, 16)
                tv = tb[i, sl]
                for b in range(B):
                    xb[b, i, sl] = xb[b, i, sl] + tv

            start_xstore(c)
            if c + NBUF < nch:
                start_tload(c + NBUF)

        for c in range(max(0, nch - NBUF), nch):
            wait_xout(c)

    return k(x, table)


def kernel(x, table):
    return _sc_add(x, table)
